# baseline (speedup 1.0000x reference)
import functools

import jax
import jax.numpy as jnp
from jax import lax
from jax.experimental import pallas as pl
from jax.experimental.pallas import tpu as pltpu
from jax.experimental.pallas import tpu_sc as plsc

NUM_CORES = 2
NUM_SUBCORES = 16
NUM_WORKERS = NUM_CORES * NUM_SUBCORES

EMB_DIM = 64
CHUNK = 640
TR_BK = 8192


def _emb_body(idx_hbm, table_hbm, out_hbm, idx_v, rows_v, g0, g1, o0, o1,
              *, per_w, n_chunks):
    wid = lax.axis_index("s") * NUM_CORES + lax.axis_index("c")
    base = wid * per_w
    gsem = (g0, g1)
    osem = (o0, o1)

    def stage_idx(slot, g):
        pltpu.sync_copy(idx_hbm.at[pl.ds(base + g * CHUNK, CHUNK)],
                        idx_v.at[slot])

    def fire_gather(slot):
        pltpu.async_copy(table_hbm.at[idx_v.at[slot]], rows_v.at[slot],
                         gsem[slot])

    def wait_gather(slot):
        pltpu.make_async_copy(table_hbm.at[idx_v.at[slot]], rows_v.at[slot],
                              gsem[slot]).wait()

    def fire_out(slot, g):
        pltpu.async_copy(rows_v.at[slot],
                         out_hbm.at[pl.ds(base + g * CHUNK, CHUNK)],
                         osem[slot])

    def wait_out(slot):
        pltpu.make_async_copy(rows_v.at[slot],
                              out_hbm.at[pl.ds(base, CHUNK)],
                              osem[slot]).wait()

    stage_idx(0, 0)
    fire_gather(0)

    @pl.loop(0, n_chunks, step=2)
    def _(p):
        for b in range(2):
            g = p + b
            slot = b
            nslot = 1 - b

            @pl.when(g + 1 < n_chunks)
            def _():
                stage_idx(nslot, g + 1)

                @pl.when(g >= 1)
                def _():
                    wait_out(nslot)

                fire_gather(nslot)

            wait_gather(slot)
            fire_out(slot, g)

    wait_out(0)
    wait_out(1)


def _transpose_body(wt_ref, out_ref):
    xt = jnp.transpose(wt_ref[...])
    hb = TR_BK // 2
    out_ref[...] = jnp.concatenate([xt[0:hb], xt[hb:TR_BK]], axis=1)


def _relayout_body(rows_ref, out_ref):
    x = rows_ref[...]
    xt = jnp.transpose(x)
    out_ref[0, :, 0:8192] = xt[0:64, :]
    out_ref[0, :, 8192:16384] = xt[64:128, :]


def kernel(token_ids, weight):
    b, t = token_ids.shape
    n = b * t
    assert n % (NUM_WORKERS * CHUNK) == 0
    per_w = n // NUM_WORKERS
    n_chunks = per_w // CHUNK
    assert n_chunks % 2 == 0

    half = b // 2
    tid_t = token_ids.T.astype(jnp.int32)
    idx_flat = tid_t.reshape(t, 2, half).transpose(0, 2, 1).reshape(n)
    hb = TR_BK // 2
    l = idx_flat % TR_BK
    idx_flat = (idx_flat - l) + 2 * (l % hb) + l // hb

    v, d = weight.shape
    wt = weight.T
    n_cb = pl.cdiv(v, TR_BK)
    table2 = pl.pallas_call(
        _transpose_body,
        grid=(n_cb,),
        in_specs=[pl.BlockSpec((d, TR_BK), lambda i: (0, i))],
        out_specs=pl.BlockSpec((hb, 2 * EMB_DIM), lambda i: (i, 0)),
        out_shape=jax.ShapeDtypeStruct((n_cb * hb, 2 * EMB_DIM), jnp.float32),
    )(wt)

    mesh = plsc.VectorSubcoreMesh(
        core_axis_name="c", subcore_axis_name="s",
        num_cores=NUM_CORES, num_subcores=NUM_SUBCORES,
    )
    emb = pl.kernel(
        functools.partial(_emb_body, per_w=per_w, n_chunks=n_chunks),
        out_type=jax.ShapeDtypeStruct((n, EMB_DIM), jnp.float32),
        mesh=mesh,
        scratch_types=[
            pltpu.VMEM((2, CHUNK), jnp.int32),
            pltpu.VMEM((2, CHUNK, EMB_DIM), jnp.float32),
            pltpu.SemaphoreType.DMA,
            pltpu.SemaphoreType.DMA,
            pltpu.SemaphoreType.DMA,
            pltpu.SemaphoreType.DMA,
        ],
        compiler_params=pltpu.CompilerParams(use_tc_tiling_on_sc=False),
    )
    table_lin = table2.reshape(n_cb * TR_BK, d)

    rows = emb(idx_flat, table_lin)
    rows128 = rows.reshape(n // 2, 2 * EMB_DIM)

    out3 = pl.pallas_call(
        _relayout_body,
        grid=(t,),
        in_specs=[pl.BlockSpec((half, 2 * EMB_DIM), lambda i: (i, 0))],
        out_specs=pl.BlockSpec((1, EMB_DIM, b), lambda i: (i, 0, 0)),
        out_shape=jax.ShapeDtypeStruct((t, EMB_DIM, b), jnp.float32),
    )(rows128)

    return jnp.transpose(out3, (2, 0, 1))


# speedup vs baseline: 1.0023x; 1.0023x over previous
import functools

import jax
import jax.numpy as jnp
from jax import lax
from jax.experimental import pallas as pl
from jax.experimental.pallas import tpu as pltpu
from jax.experimental.pallas import tpu_sc as plsc

NUM_CORES = 2
NUM_SUBCORES = 16
NUM_WORKERS = NUM_CORES * NUM_SUBCORES

EMB_DIM = 64
CHUNK = 512
TR_BK = 8192


def _emb_body(idx_hbm, table_hbm, out_hbm, idx_v, rows_v, g0, g1, o0, o1,
              *, per_w, n_chunks):
    wid = lax.axis_index("s") * NUM_CORES + lax.axis_index("c")
    base = wid * per_w
    gsem = (g0, g1)
    osem = (o0, o1)

    def stage_idx(slot, g):
        pltpu.sync_copy(idx_hbm.at[pl.ds(base + g * CHUNK, CHUNK)],
                        idx_v.at[slot])

    def fire_gather(slot):
        pltpu.async_copy(table_hbm.at[idx_v.at[slot]], rows_v.at[slot],
                         gsem[slot])

    def wait_gather(slot):
        pltpu.make_async_copy(table_hbm.at[idx_v.at[slot]], rows_v.at[slot],
                              gsem[slot]).wait()

    def fire_out(slot, g):
        pltpu.async_copy(rows_v.at[slot],
                         out_hbm.at[pl.ds(base + g * CHUNK, CHUNK)],
                         osem[slot])

    def wait_out(slot):
        pltpu.make_async_copy(rows_v.at[slot],
                              out_hbm.at[pl.ds(base, CHUNK)],
                              osem[slot]).wait()

    stage_idx(0, 0)
    fire_gather(0)

    @pl.loop(0, n_chunks, step=2)
    def _(p):
        for b in range(2):
            g = p + b
            slot = b
            nslot = 1 - b

            @pl.when(g + 1 < n_chunks)
            def _():
                stage_idx(nslot, g + 1)

                @pl.when(g >= 1)
                def _():
                    wait_out(nslot)

                fire_gather(nslot)

            wait_gather(slot)
            fire_out(slot, g)

    wait_out(0)
    wait_out(1)


def _transpose_body(wt_ref, out_ref):
    xt = jnp.transpose(wt_ref[...])
    hb = TR_BK // 2
    out_ref[...] = jnp.concatenate([xt[0:hb], xt[hb:TR_BK]], axis=1)


def _relayout_body(rows_ref, out_ref):
    x = rows_ref[...]
    xt = jnp.transpose(x)
    out_ref[0, :, 0:8192] = xt[0:64, :]
    out_ref[0, :, 8192:16384] = xt[64:128, :]


def kernel(token_ids, weight):
    b, t = token_ids.shape
    n = b * t
    assert n % (NUM_WORKERS * CHUNK) == 0
    per_w = n // NUM_WORKERS
    n_chunks = per_w // CHUNK
    assert n_chunks % 2 == 0

    half = b // 2
    tid_t = token_ids.T.astype(jnp.int32)
    idx_flat = tid_t.reshape(t, 2, half).transpose(0, 2, 1).reshape(n)
    hb = TR_BK // 2
    l = idx_flat % TR_BK
    idx_flat = (idx_flat - l) + 2 * (l % hb) + l // hb

    v, d = weight.shape
    wt = weight.T
    n_cb = pl.cdiv(v, TR_BK)
    table2 = pl.pallas_call(
        _transpose_body,
        grid=(n_cb,),
        in_specs=[pl.BlockSpec((d, TR_BK), lambda i: (0, i))],
        out_specs=pl.BlockSpec((hb, 2 * EMB_DIM), lambda i: (i, 0)),
        out_shape=jax.ShapeDtypeStruct((n_cb * hb, 2 * EMB_DIM), jnp.float32),
    )(wt)

    mesh = plsc.VectorSubcoreMesh(
        core_axis_name="c", subcore_axis_name="s",
        num_cores=NUM_CORES, num_subcores=NUM_SUBCORES,
    )
    emb = pl.kernel(
        functools.partial(_emb_body, per_w=per_w, n_chunks=n_chunks),
        out_type=jax.ShapeDtypeStruct((n, EMB_DIM), jnp.float32),
        mesh=mesh,
        scratch_types=[
            pltpu.VMEM((2, CHUNK), jnp.int32),
            pltpu.VMEM((2, CHUNK, EMB_DIM), jnp.float32),
            pltpu.SemaphoreType.DMA,
            pltpu.SemaphoreType.DMA,
            pltpu.SemaphoreType.DMA,
            pltpu.SemaphoreType.DMA,
        ],
        compiler_params=pltpu.CompilerParams(use_tc_tiling_on_sc=False),
    )
    table_lin = table2.reshape(n_cb * TR_BK, d)

    rows = emb(idx_flat, table_lin)
    rows128 = rows.reshape(n // 2, 2 * EMB_DIM)

    out3 = pl.pallas_call(
        _relayout_body,
        grid=(t,),
        in_specs=[pl.BlockSpec((half, 2 * EMB_DIM), lambda i: (i, 0))],
        out_specs=pl.BlockSpec((1, EMB_DIM, b), lambda i: (i, 0, 0)),
        out_shape=jax.ShapeDtypeStruct((t, EMB_DIM, b), jnp.float32),
    )(rows128)

    return jnp.transpose(out3, (2, 0, 1))
